# Initial kernel scaffold; baseline (speedup 1.0000x reference)
import jax
import jax.numpy as jnp
from jax.experimental import pallas as pl


def kernel(node_feats, gnn_W, gnn_b, mlp_W, mlp_b, ini_embeds, edge_weight, edge_index):
    raise NotImplementedError("write your pallas kernel here")


# speedup vs baseline: 3.3935x; 3.3935x over previous
import functools

import jax
import jax.numpy as jnp
from jax import lax
from jax.experimental import pallas as pl
from jax.experimental.pallas import tpu as pltpu
from jax.experimental.pallas import tpu_sc as plsc

_NC = 2
_NS = 16
_LANES = 16


def _spmm_sc(h, src, dst, w, n, d, npad):
    e = src.shape[0]
    nw = _NC * _NS
    epw = e // nw
    blk = 80
    nblk = epw // blk
    rpt = npad // _NS
    zr = 128
    assert epw * nw == e and nblk * blk == epw and rpt * _NS == npad
    assert rpt % zr == 0 and rpt % 8 == 0 and d % _LANES == 0
    nch = d // _LANES

    mesh = plsc.VectorSubcoreMesh(
        core_axis_name="c", subcore_axis_name="s",
        num_cores=_NC, num_subcores=_NS)

    @functools.partial(
        pl.kernel,
        out_type=jax.ShapeDtypeStruct((_NC, npad, d), jnp.float32),
        mesh=mesh,
        compiler_params=pltpu.CompilerParams(needs_layout_passes=False),
        scratch_types=[
            pltpu.VMEM((blk,), jnp.int32),
            pltpu.VMEM((blk,), jnp.int32),
            pltpu.VMEM((blk,), jnp.float32),
            pltpu.VMEM((blk, d), jnp.float32),
            pltpu.VMEM((zr, d), jnp.float32),
            pltpu.VMEM_SHARED((npad, d), jnp.float32),
            pltpu.SemaphoreType.DMA,
        ],
    )
    def spmm(h_hbm, src_hbm, dst_hbm, w_hbm, out_hbm,
             sidx, didx, wref, rows, zbuf, acc, sem):
        c = lax.axis_index("c")
        s = lax.axis_index("s")
        wid = c * _NS + s

        zero16 = jnp.zeros((_LANES,), jnp.float32)

        def zero_row(i, carry):
            for ch in range(nch):
                zbuf[i, pl.ds(ch * _LANES, _LANES)] = zero16
            return carry

        lax.fori_loop(0, zr, zero_row, 0)
        for t in range(rpt // zr):
            pltpu.sync_copy(zbuf, acc.at[pl.ds(s * rpt + t * zr, zr)])
        plsc.subcore_barrier()

        ebase = wid * epw

        def block(j, carry):
            b = ebase + j * blk
            pltpu.sync_copy(src_hbm.at[pl.ds(b, blk)], sidx)
            pltpu.sync_copy(dst_hbm.at[pl.ds(b, blk)], didx)
            pltpu.sync_copy(w_hbm.at[pl.ds(b, blk)], wref)
            pltpu.async_copy(h_hbm.at[sidx], rows, sem).wait()

            def edge(i, carry2):
                wb = plsc.load_gather(wref, [jnp.full((_LANES,), i, jnp.int32)])
                for ch in range(nch):
                    sl = pl.ds(ch * _LANES, _LANES)
                    rows[i, sl] = rows[i, sl] * wb
                return carry2

            lax.fori_loop(0, blk, edge, 0)
            pltpu.sync_copy(rows, acc.at[didx], add=True)
            return carry

        lax.fori_loop(0, nblk, block, 0)
        plsc.subcore_barrier()

        pltpu.sync_copy(acc.at[pl.ds(s * rpt, rpt)],
                        out_hbm.at[c, pl.ds(s * rpt, rpt)])

    return spmm(h, src, dst, w)


def _dense_layer(p0, p1, W, b, n, d, br=1000):
    def body(p0_ref, p1_ref, w_ref, b_ref, o_ref):
        s = p0_ref[...] + p1_ref[...]
        y = jnp.dot(s, w_ref[...], preferred_element_type=jnp.float32) + b_ref[...]
        o_ref[...] = jnp.where(y > 0, y, jnp.exp(y) - 1.0)

    return pl.pallas_call(
        body,
        grid=(n // br,),
        in_specs=[
            pl.BlockSpec((br, d), lambda i: (i, 0)),
            pl.BlockSpec((br, d), lambda i: (i, 0)),
            pl.BlockSpec((d, d), lambda i: (0, 0)),
            pl.BlockSpec((1, d), lambda i: (0, 0)),
        ],
        out_specs=pl.BlockSpec((br, d), lambda i: (i, 0)),
        out_shape=jax.ShapeDtypeStruct((n, d), jnp.float32),
    )(p0, p1, W, b.reshape(1, d))


def _final(h0, h1, h2, mw0, mw1, mb0, mb1, ini, n, d, br=1000):
    def body(h0_ref, h1_ref, h2_ref, mw0_ref, mw1_ref, mb0_ref, mb1_ref,
             ini_ref, o_ref):
        z = (h0_ref[...] + h1_ref[...] + h2_ref[...]) / 3.0
        t = jnp.dot(z, mw0_ref[...], preferred_element_type=jnp.float32) + mb0_ref[...]
        t = jnp.maximum(t, 0.0)
        t = jnp.dot(t, mw1_ref[...], preferred_element_type=jnp.float32) + mb1_ref[...]
        t = jnp.maximum(t, 0.0)
        o_ref[...] = ini_ref[...] + t

    row_spec = pl.BlockSpec((br, d), lambda i: (i, 0))
    mat_spec = pl.BlockSpec((d, d), lambda i: (0, 0))
    vec_spec = pl.BlockSpec((1, d), lambda i: (0, 0))
    return pl.pallas_call(
        body,
        grid=(n // br,),
        in_specs=[row_spec, row_spec, row_spec, mat_spec, mat_spec,
                  vec_spec, vec_spec, row_spec],
        out_specs=row_spec,
        out_shape=jax.ShapeDtypeStruct((n, d), jnp.float32),
    )(h0, h1, h2, mw0, mw1, mb0, mb1, ini)


def kernel(node_feats, gnn_W, gnn_b, mlp_W, mlp_b, ini_embeds, edge_weight,
           edge_index):
    n, d = node_feats.shape
    src = edge_index[0].astype(jnp.int32)
    dst = edge_index[1].astype(jnp.int32)
    w = edge_weight.astype(jnp.float32)

    npad = ((n + 2047) // 2048) * 2048

    h = node_feats
    layer_outs = [h]
    for l in range(gnn_W.shape[0]):
        p = _spmm_sc(h, src, dst, w, n, d, npad)
        h = _dense_layer(p[0], p[1], gnn_W[l], gnn_b[l], n, d)
        layer_outs.append(h)

    return _final(layer_outs[0], layer_outs[1], layer_outs[2],
                  mlp_W[0], mlp_W[1], mlp_b[0], mlp_b[1], ini_embeds, n, d)


# speedup vs baseline: 9.5637x; 2.8182x over previous
import functools

import jax
import jax.numpy as jnp
from jax import lax
from jax.experimental import pallas as pl
from jax.experimental.pallas import tpu as pltpu
from jax.experimental.pallas import tpu_sc as plsc

_NC = 2
_NS = 16
_LANES = 16


def _spmm_sc(h, src, dst, w, n, d, npad):
    e = src.shape[0]
    nw = _NC * _NS
    epw = e // nw
    blk = 40
    nblk = epw // blk
    nbuf = 5
    rpt = npad // _NS
    assert epw * nw == e and nblk * blk == epw and rpt * _NS == npad
    assert nblk % nbuf == 0 and nblk >= 2 * nbuf
    assert rpt % blk == 0 and rpt % 8 == 0 and d % _LANES == 0
    nch = d // _LANES

    mesh = plsc.VectorSubcoreMesh(
        core_axis_name="c", subcore_axis_name="s",
        num_cores=_NC, num_subcores=_NS)

    @functools.partial(
        pl.kernel,
        out_type=jax.ShapeDtypeStruct((_NC, npad, d), jnp.float32),
        mesh=mesh,
        compiler_params=pltpu.CompilerParams(needs_layout_passes=False),
        scratch_types=[
            [pltpu.VMEM((blk,), jnp.int32) for _ in range(nbuf)],
            [pltpu.VMEM((blk,), jnp.int32) for _ in range(nbuf)],
            [pltpu.VMEM((blk,), jnp.float32) for _ in range(nbuf)],
            [pltpu.VMEM((blk, d), jnp.float32) for _ in range(nbuf)],
            pltpu.VMEM_SHARED((npad, d), jnp.float32),
            [pltpu.SemaphoreType.DMA for _ in range(nbuf)],
            [pltpu.SemaphoreType.DMA for _ in range(nbuf)],
            [pltpu.SemaphoreType.DMA for _ in range(nbuf)],
        ],
    )
    def spmm(h_hbm, src_hbm, dst_hbm, w_hbm, out_hbm,
             sidx, didx, wvb, rows, acc, asem, gsem, ssem):
        c = lax.axis_index("c")
        s = lax.axis_index("s")
        wid = c * _NS + s
        ebase = wid * epw

        def start_a(j, b):
            sl = pl.ds(ebase + j * blk, blk)
            pltpu.async_copy(src_hbm.at[sl], sidx[b], asem[b])
            pltpu.async_copy(dst_hbm.at[sl], didx[b], asem[b])
            pltpu.async_copy(w_hbm.at[sl], wvb[b], asem[b])

        def wait_a(j, b):
            sl = pl.ds(ebase + j * blk, blk)
            pltpu.make_async_copy(src_hbm.at[sl], sidx[b], asem[b]).wait()
            pltpu.make_async_copy(dst_hbm.at[sl], didx[b], asem[b]).wait()
            pltpu.make_async_copy(w_hbm.at[sl], wvb[b], asem[b]).wait()

        def start_b(b):
            pltpu.async_copy(h_hbm.at[sidx[b]], rows[b], gsem[b])

        def wait_b(b):
            pltpu.make_async_copy(h_hbm.at[sidx[b]], rows[b], gsem[b]).wait()

        def wait_scatter(b):
            pltpu.make_async_copy(rows[b], acc.at[didx[b]], ssem[b]).wait()

        for t in range(nbuf - 1):
            start_a(t, t)

        zero16 = jnp.zeros((_LANES,), jnp.float32)

        def zero_row(i, carry):
            for ch in range(nch):
                rows[nbuf - 1][i, pl.ds(ch * _LANES, _LANES)] = zero16
            return carry

        lax.fori_loop(0, blk, zero_row, 0)
        for t in range(rpt // blk):
            pltpu.sync_copy(rows[nbuf - 1],
                            acc.at[pl.ds(s * rpt + t * blk, blk)])
        plsc.subcore_barrier()

        for t in range(nbuf - 2):
            wait_a(t, t)
            start_b(t)

        def outer(g, carry):
            for t in range(nbuf):
                jj = g * nbuf + t
                b3 = (t + 3) % nbuf
                b4 = (t + 4) % nbuf

                wait_b(t)

                @plsc.parallel_loop(0, blk, 1, unroll=4)
                def scale(i):
                    wb = plsc.load_gather(
                        wvb[t], [jnp.full((_LANES,), i, jnp.int32)])
                    for ch in range(nch):
                        sl = pl.ds(ch * _LANES, _LANES)
                        rows[t][i, sl] = rows[t][i, sl] * wb

                pltpu.async_copy(rows[t], acc.at[didx[t]], ssem[t], add=True)

                @pl.when(jj + 3 < nblk)
                def _advance_b():
                    wait_a(jj + 3, b3)
                    start_b(b3)

                @pl.when(jj + 4 < nblk)
                def _advance_a():
                    @pl.when(jj >= 1)
                    def _drain_scatter():
                        wait_scatter(b4)

                    start_a(jj + 4, b4)

            return carry

        lax.fori_loop(0, nblk // nbuf, outer, 0)
        for t in range(nbuf):
            wait_scatter(t)
        plsc.subcore_barrier()

        pltpu.sync_copy(acc.at[pl.ds(s * rpt, rpt)],
                        out_hbm.at[c, pl.ds(s * rpt, rpt)])

    return spmm(h, src, dst, w)


def _dense_layer(p0, p1, W, b, n, d, br=1000):
    def body(p0_ref, p1_ref, w_ref, b_ref, o_ref):
        s = p0_ref[...] + p1_ref[...]
        y = jnp.dot(s, w_ref[...], preferred_element_type=jnp.float32) + b_ref[...]
        o_ref[...] = jnp.where(y > 0, y, jnp.exp(y) - 1.0)

    return pl.pallas_call(
        body,
        grid=(n // br,),
        in_specs=[
            pl.BlockSpec((br, d), lambda i: (i, 0)),
            pl.BlockSpec((br, d), lambda i: (i, 0)),
            pl.BlockSpec((d, d), lambda i: (0, 0)),
            pl.BlockSpec((1, d), lambda i: (0, 0)),
        ],
        out_specs=pl.BlockSpec((br, d), lambda i: (i, 0)),
        out_shape=jax.ShapeDtypeStruct((n, d), jnp.float32),
    )(p0, p1, W, b.reshape(1, d))


def _final(h0, h1, h2, mw0, mw1, mb0, mb1, ini, n, d, br=1000):
    def body(h0_ref, h1_ref, h2_ref, mw0_ref, mw1_ref, mb0_ref, mb1_ref,
             ini_ref, o_ref):
        z = (h0_ref[...] + h1_ref[...] + h2_ref[...]) / 3.0
        t = jnp.dot(z, mw0_ref[...], preferred_element_type=jnp.float32) + mb0_ref[...]
        t = jnp.maximum(t, 0.0)
        t = jnp.dot(t, mw1_ref[...], preferred_element_type=jnp.float32) + mb1_ref[...]
        t = jnp.maximum(t, 0.0)
        o_ref[...] = ini_ref[...] + t

    row_spec = pl.BlockSpec((br, d), lambda i: (i, 0))
    mat_spec = pl.BlockSpec((d, d), lambda i: (0, 0))
    vec_spec = pl.BlockSpec((1, d), lambda i: (0, 0))
    return pl.pallas_call(
        body,
        grid=(n // br,),
        in_specs=[row_spec, row_spec, row_spec, mat_spec, mat_spec,
                  vec_spec, vec_spec, row_spec],
        out_specs=row_spec,
        out_shape=jax.ShapeDtypeStruct((n, d), jnp.float32),
    )(h0, h1, h2, mw0, mw1, mb0, mb1, ini)


def kernel(node_feats, gnn_W, gnn_b, mlp_W, mlp_b, ini_embeds, edge_weight,
           edge_index):
    n, d = node_feats.shape
    src = edge_index[0].astype(jnp.int32)
    dst = edge_index[1].astype(jnp.int32)
    w = edge_weight.astype(jnp.float32)

    npad = ((n + 2047) // 2048) * 2048

    h = node_feats
    layer_outs = [h]
    for l in range(gnn_W.shape[0]):
        p = _spmm_sc(h, src, dst, w, n, d, npad)
        h = _dense_layer(p[0], p[1], gnn_W[l], gnn_b[l], n, d)
        layer_outs.append(h)

    return _final(layer_outs[0], layer_outs[1], layer_outs[2],
                  mlp_W[0], mlp_W[1], mlp_b[0], mlp_b[1], ini_embeds, n, d)


# speedup vs baseline: 11.0183x; 1.1521x over previous
import functools

import jax
import jax.numpy as jnp
from jax import lax
from jax.experimental import pallas as pl
from jax.experimental.pallas import tpu as pltpu
from jax.experimental.pallas import tpu_sc as plsc

_NC = 2
_NS = 16
_LANES = 16


def _spmm_sc(h, src, dst, w, n, d, npad):
    e = src.shape[0]
    nw = _NC * _NS
    epw = e // nw
    chk = 2000
    nchk = epw // chk
    blk = 40
    nbpc = chk // blk
    nbuf = 5
    rpt = npad // _NS
    assert epw * nw == e and nchk * chk == epw and nbpc * blk == chk
    assert nbpc % nbuf == 0 and nbpc >= 2 * nbuf and nchk >= 2
    assert rpt % blk == 0 and rpt % 8 == 0 and d % _LANES == 0
    nch = d // _LANES

    mesh = plsc.VectorSubcoreMesh(
        core_axis_name="c", subcore_axis_name="s",
        num_cores=_NC, num_subcores=_NS)

    @functools.partial(
        pl.kernel,
        out_type=jax.ShapeDtypeStruct((_NC, npad, d), jnp.float32),
        mesh=mesh,
        compiler_params=pltpu.CompilerParams(needs_layout_passes=False),
        scratch_types=[
            [pltpu.VMEM((chk,), jnp.int32) for _ in range(2)],
            [pltpu.VMEM((chk,), jnp.int32) for _ in range(2)],
            [pltpu.VMEM((chk,), jnp.float32) for _ in range(2)],
            [pltpu.VMEM((blk, d), jnp.float32) for _ in range(nbuf)],
            pltpu.VMEM_SHARED((npad, d), jnp.float32),
            [pltpu.SemaphoreType.DMA for _ in range(2)],
            [pltpu.SemaphoreType.DMA for _ in range(nbuf)],
            [pltpu.SemaphoreType.DMA for _ in range(nbuf)],
        ],
    )
    def spmm(h_hbm, src_hbm, dst_hbm, w_hbm, out_hbm,
             scb, dcb, wcb, rows, acc, csem, gsem, ssem):
        c = lax.axis_index("c")
        s = lax.axis_index("s")
        wid = c * _NS + s
        ebase = wid * epw

        def fetch_chunk(k, cb):
            sl = pl.ds(ebase + k * chk, chk)
            pltpu.async_copy(src_hbm.at[sl], scb[cb], csem[cb])
            pltpu.async_copy(dst_hbm.at[sl], dcb[cb], csem[cb])
            pltpu.async_copy(w_hbm.at[sl], wcb[cb], csem[cb])

        def wait_chunk(k, cb):
            sl = pl.ds(ebase + k * chk, chk)
            pltpu.make_async_copy(src_hbm.at[sl], scb[cb], csem[cb]).wait()
            pltpu.make_async_copy(dst_hbm.at[sl], dcb[cb], csem[cb]).wait()
            pltpu.make_async_copy(w_hbm.at[sl], wcb[cb], csem[cb]).wait()

        def start_gather(cb, j, b):
            idx = scb[cb].at[pl.ds(j * blk, blk)]
            pltpu.async_copy(h_hbm.at[idx], rows[b], gsem[b])

        def wait_gather(cb, j, b):
            idx = scb[cb].at[pl.ds(j * blk, blk)]
            pltpu.make_async_copy(h_hbm.at[idx], rows[b], gsem[b]).wait()

        def start_scatter(cb, j, b):
            idx = dcb[cb].at[pl.ds(j * blk, blk)]
            pltpu.async_copy(rows[b], acc.at[idx], ssem[b], add=True)

        def wait_scatter(cb, j, b):
            idx = dcb[cb].at[pl.ds(j * blk, blk)]
            pltpu.make_async_copy(rows[b], acc.at[idx], ssem[b]).wait()

        fetch_chunk(0, 0)

        zero16 = jnp.zeros((_LANES,), jnp.float32)

        def zero_row(i, carry):
            for ch in range(nch):
                rows[nbuf - 1][i, pl.ds(ch * _LANES, _LANES)] = zero16
            return carry

        lax.fori_loop(0, blk, zero_row, 0)
        for t in range(rpt // blk):
            pltpu.sync_copy(rows[nbuf - 1],
                            acc.at[pl.ds(s * rpt + t * blk, blk)])
        plsc.subcore_barrier()

        wait_chunk(0, 0)
        fetch_chunk(1, 1)

        for k in range(nchk):
            cb = k % 2

            if k > 0:
                wait_chunk(k, cb)

            for t in range(nbuf - 1):
                start_gather(cb, t, t)

            def body(g, carry):
                for t in range(nbuf):
                    j = g * nbuf + t

                    wait_gather(cb, j, t)

                    @plsc.parallel_loop(0, blk, 1, unroll=4)
                    def scale(i):
                        wb = plsc.load_gather(
                            wcb[cb],
                            [jnp.full((_LANES,), j * blk + i, jnp.int32)])
                        for ch in range(nch):
                            sl = pl.ds(ch * _LANES, _LANES)
                            rows[t][i, sl] = rows[t][i, sl] * wb

                    start_scatter(cb, j, t)

                    bn = (t + nbuf - 1) % nbuf

                    @pl.when(j + nbuf - 1 < nbpc)
                    def _advance_gather():
                        @pl.when(jnp.bool_(j >= 1))
                        def _drain_prev_scatter():
                            wait_scatter(cb, j - 1, bn)

                        start_gather(cb, j + nbuf - 1, bn)

                return carry

            lax.fori_loop(0, nbpc // nbuf, body, 0)

            for t in range(nbuf):
                wait_scatter(cb, nbpc - nbuf + t, t)

            if k + 2 < nchk:
                fetch_chunk(k + 2, cb)

        plsc.subcore_barrier()

        pltpu.sync_copy(acc.at[pl.ds(s * rpt, rpt)],
                        out_hbm.at[c, pl.ds(s * rpt, rpt)])

    return spmm(h, src, dst, w)


def _dense_layer(p0, p1, W, b, n, d, br=1000):
    def body(p0_ref, p1_ref, w_ref, b_ref, o_ref):
        s = p0_ref[...] + p1_ref[...]
        y = jnp.dot(s, w_ref[...], preferred_element_type=jnp.float32) + b_ref[...]
        o_ref[...] = jnp.where(y > 0, y, jnp.exp(y) - 1.0)

    return pl.pallas_call(
        body,
        grid=(n // br,),
        in_specs=[
            pl.BlockSpec((br, d), lambda i: (i, 0)),
            pl.BlockSpec((br, d), lambda i: (i, 0)),
            pl.BlockSpec((d, d), lambda i: (0, 0)),
            pl.BlockSpec((1, d), lambda i: (0, 0)),
        ],
        out_specs=pl.BlockSpec((br, d), lambda i: (i, 0)),
        out_shape=jax.ShapeDtypeStruct((n, d), jnp.float32),
    )(p0, p1, W, b.reshape(1, d))


def _dense2_final(p0, p1, W, b, h0, h1, mw0, mw1, mb0, mb1, ini, n, d,
                  br=1000):
    def body(p0_ref, p1_ref, w_ref, b_ref, h0_ref, h1_ref, mw0_ref, mw1_ref,
             mb0_ref, mb1_ref, ini_ref, o_ref):
        sacc = p0_ref[...] + p1_ref[...]
        y = jnp.dot(sacc, w_ref[...], preferred_element_type=jnp.float32) + b_ref[...]
        h2 = jnp.where(y > 0, y, jnp.exp(y) - 1.0)
        z = (h0_ref[...] + h1_ref[...] + h2) / 3.0
        t = jnp.dot(z, mw0_ref[...], preferred_element_type=jnp.float32) + mb0_ref[...]
        t = jnp.maximum(t, 0.0)
        t = jnp.dot(t, mw1_ref[...], preferred_element_type=jnp.float32) + mb1_ref[...]
        t = jnp.maximum(t, 0.0)
        o_ref[...] = ini_ref[...] + t

    row_spec = pl.BlockSpec((br, d), lambda i: (i, 0))
    mat_spec = pl.BlockSpec((d, d), lambda i: (0, 0))
    vec_spec = pl.BlockSpec((1, d), lambda i: (0, 0))
    return pl.pallas_call(
        body,
        grid=(n // br,),
        in_specs=[row_spec, row_spec, mat_spec, vec_spec, row_spec, row_spec,
                  mat_spec, mat_spec, vec_spec, vec_spec, row_spec],
        out_specs=row_spec,
        out_shape=jax.ShapeDtypeStruct((n, d), jnp.float32),
    )(p0, p1, W, b.reshape(1, d), h0, h1, mw0, mw1, mb0.reshape(1, d),
      mb1.reshape(1, d), ini)


def kernel(node_feats, gnn_W, gnn_b, mlp_W, mlp_b, ini_embeds, edge_weight,
           edge_index):
    n, d = node_feats.shape
    src = edge_index[0].astype(jnp.int32)
    dst = edge_index[1].astype(jnp.int32)
    w = edge_weight.astype(jnp.float32)

    npad = ((n + 2047) // 2048) * 2048

    h0 = node_feats
    p = _spmm_sc(h0, src, dst, w, n, d, npad)
    h1 = _dense_layer(p[0], p[1], gnn_W[0], gnn_b[0], n, d)
    p = _spmm_sc(h1, src, dst, w, n, d, npad)
    return _dense2_final(p[0], p[1], gnn_W[1], gnn_b[1], h0, h1,
                         mlp_W[0], mlp_W[1], mlp_b[0], mlp_b[1],
                         ini_embeds, n, d)
